# baseline (speedup 1.0000x reference)
import jax
import jax.numpy as jnp
from jax.experimental import pallas as pl
from jax.experimental.pallas import tpu as pltpu

Q = 512
P = 8
S = 8
PS = P * S
N = 16
I = 2048


def _mixmil_kernel(qmu_ref, qls_ref, eps_ref, x_ref, out_ref,
                   w_scr, b_scr, u_scr):
    n = pl.program_id(0)

    @pl.when(n == 0)
    def _prep():
        beta = qmu_ref[...] + jnp.exp(qls_ref[...]) * eps_ref[...]
        beta_u = beta[:Q]
        beta_z = beta[Q:]
        z2 = beta_z * beta_z
        b_row = jnp.sqrt(jnp.mean(z2, axis=0, keepdims=True))
        eta = beta_z / b_row
        w_scr[...] = jnp.concatenate([beta_u, eta], axis=1)
        ones_col = jnp.ones((Q, 1), dtype=jnp.float32)
        b_scr[...] = jnp.sqrt(
            jax.lax.dot_general(z2, ones_col, (((0,), (0,)), ((), ())),
                                preferred_element_type=jnp.float32) / Q)
        u_scr[...] = jnp.zeros((PS, N), dtype=jnp.float32)

    x = x_ref[0]
    y = jax.lax.dot_general(w_scr[...], x, (((0,), (1,)), ((), ())),
                            preferred_element_type=jnp.float32)
    a = y[:PS, :]
    t = y[PS:, :]
    m = jnp.max(a, axis=1, keepdims=True)
    e = jnp.exp(a - m)
    denom = jnp.sum(e, axis=1, keepdims=True)
    num = jnp.sum(e * t, axis=1, keepdims=True)
    lane = jax.lax.broadcasted_iota(jnp.int32, (PS, N), 1)
    u_scr[...] += jnp.where(lane == n, num / denom, 0.0)

    @pl.when(n == N - 1)
    def _final():
        u = u_scr[...]
        mean = jnp.mean(u, axis=1, keepdims=True)
        d = u - mean
        std = jnp.sqrt(jnp.sum(d * d, axis=1, keepdims=True) / (N - 1))
        out_ref[...] = jnp.transpose(b_scr[...] * d / std)


def kernel(Xs, q_mu, q_log_sigma, eps):
    qmu64 = jnp.repeat(q_mu, S, axis=1)
    qls64 = jnp.repeat(q_log_sigma, S, axis=1)
    eps64 = eps.reshape(2 * Q, PS)

    uT = pl.pallas_call(
        _mixmil_kernel,
        grid=(N,),
        in_specs=[
            pl.BlockSpec((2 * Q, PS), lambda n: (0, 0)),
            pl.BlockSpec((2 * Q, PS), lambda n: (0, 0)),
            pl.BlockSpec((2 * Q, PS), lambda n: (0, 0)),
            pl.BlockSpec((1, I, Q), lambda n: (n, 0, 0)),
        ],
        out_specs=pl.BlockSpec((N, PS), lambda n: (0, 0)),
        out_shape=jax.ShapeDtypeStruct((N, PS), jnp.float32),
        scratch_shapes=[
            pltpu.VMEM((Q, 2 * PS), jnp.float32),
            pltpu.VMEM((PS, 1), jnp.float32),
            pltpu.VMEM((PS, N), jnp.float32),
        ],
    )(qmu64, qls64, eps64, Xs)
    return uT.reshape(N, P, S)
